# baseline (speedup 1.0000x reference)
import functools

import jax
import jax.numpy as jnp
from jax import lax
from jax.experimental import pallas as pl
from jax.experimental.pallas import tpu as pltpu
from jax.experimental.pallas import tpu_sc as plsc

NC, NS, L = 2, 16, 16
NW = NC * NS
B = 16384
D = 64
W128 = 128
BPW = B // NW
CHUNK = 128
NCH = BPW // CHUNK
GROUPS = CHUNK // L

_mesh = plsc.VectorSubcoreMesh(
    core_axis_name="c", subcore_axis_name="s", num_cores=NC, num_subcores=NS)


@functools.partial(
    pl.kernel,
    out_type=jax.ShapeDtypeStruct((B,), jnp.float32),
    mesh=_mesh,
    compiler_params=pltpu.CompilerParams(
        needs_layout_passes=False, use_tc_tiling_on_sc=True),
    scratch_types=[
        pltpu.VMEM((NCH, CHUNK), jnp.int32),
        pltpu.VMEM((NCH, CHUNK), jnp.int32),
        pltpu.VMEM((CHUNK, W128), jnp.float32),
        pltpu.VMEM((CHUNK, W128), jnp.float32),
        pltpu.VMEM((96,), jnp.float32),
        pltpu.VMEM((L * L,), jnp.float32),
        pltpu.VMEM((BPW,), jnp.float32),
        pltpu.SemaphoreType.DMA,
    ],
)
def _afmp_sc(tab_hbm, ia_hbm, ib_hbm, w_hbm, out_hbm,
             ia_v, ib_v, ra_v, rb_v, w_v, m_v, o_v, sem):
    wid = lax.axis_index("s") * NC + lax.axis_index("c")
    base = wid * BPW
    pltpu.sync_copy(ia_hbm.at[wid], ia_v)
    pltpu.sync_copy(ib_hbm.at[wid], ib_v)
    pltpu.sync_copy(w_hbm, w_v)
    wv = [w_v[pl.ds(c * L, L)] for c in range(D // L)]
    wmask = w_v[pl.ds(D, L)]
    b0v = w_v[pl.ds(D + L, L)]
    iota = lax.iota(jnp.int32, L)

    for j in range(NCH):
        cpa = pltpu.async_copy(tab_hbm.at[ia_v.at[j]], ra_v, sem)
        cpb = pltpu.async_copy(tab_hbm.at[ib_v.at[j]], rb_v, sem)
        cpa.wait()
        cpb.wait()

        def group(g, _):
            rbase = g * L
            for r in range(L):
                row = rbase + r
                acc = ra_v[row, pl.ds(0, L)] * rb_v[row, pl.ds(0, L)] * wv[0]
                for c in range(1, D // L):
                    acc = acc + (ra_v[row, pl.ds(c * L, L)]
                                 * rb_v[row, pl.ds(c * L, L)] * wv[c])
                acc = acc + (ra_v[row, pl.ds(D, L)]
                             + rb_v[row, pl.ds(D, L)]) * wmask
                m_v[pl.ds(r * L, L)] = acc
            res = plsc.load_gather(m_v, [iota * L])
            for kcol in range(1, L):
                res = res + plsc.load_gather(m_v, [iota * L + kcol])
            x = res + b0v
            o_v[pl.ds(j * CHUNK + rbase, L)] = 1.0 / (1.0 + jnp.exp(-x))
            return 0

        lax.fori_loop(0, GROUPS, group, 0)

    pltpu.sync_copy(o_v, out_hbm.at[pl.ds(base, BPW)])


def kernel(drug_a, drug_b, emb_table, bias_table, dense_W, dense_b):
    n_rows = emb_table.shape[0]
    aug = jnp.concatenate(
        [emb_table, bias_table,
         jnp.zeros((n_rows, W128 - D - 1), jnp.float32)], axis=1)
    ia = drug_a.astype(jnp.int32).reshape(NW, NCH, CHUNK)
    ib = drug_b.astype(jnp.int32).reshape(NW, NCH, CHUNK)
    w = dense_W[:, 0]
    wpack = jnp.concatenate([
        w[:D],
        jnp.zeros((L,), jnp.float32).at[0].set(w[D]),
        jnp.full((L,), dense_b[0], jnp.float32),
    ])
    out = _afmp_sc(aug, ia, ib, wpack)
    return out.reshape(B, 1)


# speedup vs baseline: 1.9963x; 1.9963x over previous
import functools

import jax
import jax.numpy as jnp
from jax import lax
from jax.experimental import pallas as pl
from jax.experimental.pallas import tpu as pltpu
from jax.experimental.pallas import tpu_sc as plsc

NC, NS, L = 2, 16, 16
NW = NC * NS
B = 16384
NLK = 2 * B
D = 64
NROW = 1000001
BLK = 128
NBLK_TOT = (NROW + BLK - 1) // BLK
BPT = (NBLK_TOT + NW - 1) // NW
SPAN = BPT * BLK
NBKT = 16
BKT_SPAN = 2048
HCAP = 2048
BCAP = 288
PCAP = 96
NRING = 8

BPW = B // NW
CHUNK = 128
NCH = BPW // CHUNK
GROUPS = CHUNK // L

_mesh = plsc.VectorSubcoreMesh(
    core_axis_name="c", subcore_axis_name="s", num_cores=NC, num_subcores=NS)


@functools.partial(
    pl.kernel,
    out_type=jax.ShapeDtypeStruct((NLK * D,), jnp.float32),
    mesh=_mesh,
    compiler_params=pltpu.CompilerParams(
        needs_layout_passes=False, use_tc_tiling_on_sc=True),
    scratch_types=[
        pltpu.VMEM((NLK,), jnp.int32),
        pltpu.VMEM((HCAP + L,), jnp.int32),
        pltpu.VMEM((HCAP + L,), jnp.int32),
        pltpu.VMEM((NBKT, BCAP + L), jnp.int32),
        pltpu.VMEM((NBKT, BCAP + L), jnp.int32),
        pltpu.VMEM((PCAP + L,), jnp.int32),
        pltpu.VMEM((PCAP + L,), jnp.int32),
        pltpu.VMEM((D, BLK), jnp.float32),
        pltpu.VMEM((NRING, D), jnp.float32),
        pltpu.SMEM((NBKT,), jnp.int32),
        pltpu.SemaphoreType.DMA,
        pltpu.SemaphoreType.DMA,
    ],
)
def _scan_route(tabT_hbm, lk_hbm, stage_hbm,
                lk_v, hidx_v, hslot_v, bidx_v, bslot_v, pidx_v, pslot_v,
                blk_v, ring_v, bcnt_s, bsem, rsem):
    wid = lax.axis_index("s") * NC + lax.axis_index("c")
    lo = wid * SPAN
    g0 = wid * BPT
    gmax = jnp.minimum(g0 + BPT, NBLK_TOT)
    pltpu.sync_copy(lk_hbm, lk_v)
    iota = lax.iota(jnp.int32, L)

    def scan_body(i, cur):
        v = lk_v[pl.ds(i * L, L)]
        m = (v >= lo) & (v < lo + SPAN)
        n = plsc.all_reduce_population_count(m)[0]
        plsc.store_compressed(hidx_v.at[pl.ds(cur, L)], v - lo, mask=m)
        plsc.store_compressed(hslot_v.at[pl.ds(cur, L)], iota + i * L, mask=m)
        return jnp.minimum(cur + n, HCAP)

    nhits = lax.fori_loop(0, NLK // L, scan_body, jnp.int32(0))

    for k in range(NBKT):
        def bkt_body(h, cur, k=k):
            v = hidx_v[pl.ds(h * L, L)]
            s = hslot_v[pl.ds(h * L, L)]
            valid = (iota + h * L) < nhits
            m = valid & (lax.shift_right_logical(v, 11) == k)
            n = plsc.all_reduce_population_count(m)[0]
            plsc.store_compressed(bidx_v.at[k, pl.ds(cur, L)], v, mask=m)
            plsc.store_compressed(bslot_v.at[k, pl.ds(cur, L)], s, mask=m)
            return jnp.minimum(cur + n, BCAP)

        nb = lax.fori_loop(0, (HCAP + L - 1) // L, bkt_body, jnp.int32(0))
        bcnt_s[k] = nb

    def blk_body(g, tot):
        lg = g - g0
        live = g < gmax

        @pl.when(live)
        def _():
            off = pl.multiple_of(g * BLK, BLK)
            pltpu.async_copy(tabT_hbm.at[:, pl.ds(off, BLK)], blk_v, bsem).wait()

        k = lax.shift_right_logical(lg, 4)
        ck = bcnt_s[k]
        vlo = lg * BLK

        def ext_body(h, cur):
            v = bidx_v[k, pl.ds(h * L, L)]
            s = bslot_v[k, pl.ds(h * L, L)]
            valid = (iota + h * L) < ck
            m = valid & (v >= vlo) & (v < vlo + BLK) & jnp.full((L,), live)
            n = plsc.all_reduce_population_count(m)[0]
            plsc.store_compressed(pidx_v.at[pl.ds(cur, L)], v, mask=m)
            plsc.store_compressed(pslot_v.at[pl.ds(cur, L)], s, mask=m)
            return jnp.minimum(cur + n, PCAP)

        np_ = lax.fori_loop(0, (BCAP + L) // L, ext_body, jnp.int32(0))

        def hit_body(h, tot):
            idx = pidx_v[pl.ds(h, L)][0]
            slot = pslot_v[pl.ds(h, L)][0]
            c = jnp.broadcast_to(idx & (BLK - 1), (L,))
            rs = lax.rem(tot, NRING)

            @pl.when(tot >= NRING)
            def _():
                pltpu.make_async_copy(stage_hbm.at[pl.ds(0, D)], ring_v.at[rs], rsem).wait()

            for fg in range(D // L):
                ring_v[rs, pl.ds(fg * L, L)] = plsc.load_gather(
                    blk_v, [iota + fg * L, c])
            pltpu.async_copy(ring_v.at[rs], stage_hbm.at[pl.ds(slot * D, D)], rsem)
            return tot + 1

        return lax.fori_loop(0, np_, hit_body, tot)

    tot = lax.fori_loop(g0, g0 + BPT, blk_body, jnp.int32(0))

    def drain_body(j, _):
        @pl.when(j < jnp.minimum(tot, NRING))
        def _():
            pltpu.make_async_copy(stage_hbm.at[pl.ds(0, D)], ring_v.at[j], rsem).wait()
        return 0

    lax.fori_loop(0, NRING, drain_body, 0)


@functools.partial(
    pl.kernel,
    out_type=jax.ShapeDtypeStruct((B,), jnp.float32),
    mesh=_mesh,
    compiler_params=pltpu.CompilerParams(needs_layout_passes=False),
    scratch_types=[
        pltpu.VMEM((NCH, CHUNK), jnp.int32),
        pltpu.VMEM((NCH, CHUNK), jnp.int32),
        pltpu.VMEM((CHUNK * D,), jnp.float32),
        pltpu.VMEM((CHUNK * D,), jnp.float32),
        pltpu.VMEM((CHUNK,), jnp.float32),
        pltpu.VMEM((CHUNK,), jnp.float32),
        pltpu.VMEM((96,), jnp.float32),
        pltpu.VMEM((L * L,), jnp.float32),
        pltpu.VMEM((BPW,), jnp.float32),
        pltpu.SemaphoreType.DMA,
    ],
)
def _epilogue(stage_hbm, bias_hbm, ia_hbm, ib_hbm, w_hbm, out_hbm,
              ia_v, ib_v, ra_v, rb_v, ba_v, bb_v, w_v, m_v, o_v, sem):
    wid = lax.axis_index("s") * NC + lax.axis_index("c")
    base = wid * BPW
    pltpu.sync_copy(ia_hbm.at[wid], ia_v)
    pltpu.sync_copy(ib_hbm.at[wid], ib_v)
    pltpu.sync_copy(w_hbm, w_v)
    wv = [w_v[pl.ds(c * L, L)] for c in range(D // L)]
    w64v = w_v[pl.ds(D, L)]
    b0v = w_v[pl.ds(D + L, L)]
    iota = lax.iota(jnp.int32, L)

    for j in range(NCH):
        r0 = base + j * CHUNK
        cps = [
            pltpu.async_copy(stage_hbm.at[pl.ds(r0 * D, CHUNK * D)], ra_v, sem),
            pltpu.async_copy(stage_hbm.at[pl.ds((B + r0) * D, CHUNK * D)], rb_v, sem),
            pltpu.async_copy(bias_hbm.at[ia_v.at[j]], ba_v, sem),
            pltpu.async_copy(bias_hbm.at[ib_v.at[j]], bb_v, sem),
        ]
        for cp in cps:
            cp.wait()

        def group(g, _):
            rbase = g * L
            for r in range(L):
                row = rbase + r
                acc = (ra_v[pl.ds(row * D, L)]
                       * rb_v[pl.ds(row * D, L)] * wv[0])
                for c in range(1, D // L):
                    acc = acc + (ra_v[pl.ds(row * D + c * L, L)]
                                 * rb_v[pl.ds(row * D + c * L, L)] * wv[c])
                m_v[pl.ds(r * L, L)] = acc
            res = plsc.load_gather(m_v, [iota * L])
            for kcol in range(1, L):
                res = res + plsc.load_gather(m_v, [iota * L + kcol])
            x = res + (ba_v[pl.ds(rbase, L)] + bb_v[pl.ds(rbase, L)]) * w64v + b0v
            o_v[pl.ds(j * CHUNK + rbase, L)] = 1.0 / (1.0 + jnp.exp(-x))
            return 0

        lax.fori_loop(0, GROUPS, group, 0)

    pltpu.sync_copy(o_v, out_hbm.at[pl.ds(base, BPW)])


def kernel(drug_a, drug_b, emb_table, bias_table, dense_W, dense_b):
    ia32 = drug_a.astype(jnp.int32)
    ib32 = drug_b.astype(jnp.int32)
    lk = jnp.concatenate([ia32, ib32])
    stage = _scan_route(emb_table.T, lk)
    w = dense_W[:, 0]
    wpack = jnp.concatenate([
        w[:D],
        jnp.full((L,), w[D], jnp.float32),
        jnp.full((L,), dense_b[0], jnp.float32),
    ])
    out = _epilogue(stage, bias_table[:, 0],
                    ia32.reshape(NW, NCH, CHUNK), ib32.reshape(NW, NCH, CHUNK),
                    wpack)
    return out.reshape(B, 1)


# speedup vs baseline: 3.8989x; 1.9530x over previous
import functools

import jax
import jax.numpy as jnp
from jax import lax
from jax.experimental import pallas as pl
from jax.experimental.pallas import tpu as pltpu
from jax.experimental.pallas import tpu_sc as plsc

NC, NS, L = 2, 16, 16
NW = NC * NS
B = 16384
NLK = 2 * B
D = 64
NROW = 1000001
BLK = 128
NBLK_TOT = (NROW + BLK - 1) // BLK
BPT = (NBLK_TOT + NW - 1) // NW
SPAN = BPT * BLK
NBKT = 16
BKT_SPAN = 2048
HCAP = 2048
BCAP = 288
PCAP = 96
NRING = 8
NBUF = 4

BPW = B // NW
CHUNK = 128
NCH = BPW // CHUNK
GROUPS = CHUNK // L

_mesh = plsc.VectorSubcoreMesh(
    core_axis_name="c", subcore_axis_name="s", num_cores=NC, num_subcores=NS)


@functools.partial(
    pl.kernel,
    out_type=jax.ShapeDtypeStruct((NLK * D,), jnp.float32),
    mesh=_mesh,
    compiler_params=pltpu.CompilerParams(
        needs_layout_passes=False, use_tc_tiling_on_sc=True),
    scratch_types=[
        pltpu.VMEM((NLK,), jnp.int32),
        pltpu.VMEM((HCAP + L,), jnp.int32),
        pltpu.VMEM((HCAP + L,), jnp.int32),
        pltpu.VMEM((NBKT, BCAP + L), jnp.int32),
        pltpu.VMEM((NBKT, BCAP + L), jnp.int32),
        pltpu.VMEM((PCAP + L,), jnp.int32),
        pltpu.VMEM((PCAP + L,), jnp.int32),
        pltpu.VMEM((NBUF, D, BLK), jnp.float32),
        pltpu.VMEM((NRING, D), jnp.float32),
        pltpu.SMEM((NBKT,), jnp.int32),
        pltpu.SemaphoreType.DMA,
        pltpu.SemaphoreType.DMA,
    ],
)
def _scan_route(tabT_hbm, lk_hbm, stage_hbm,
                lk_v, hidx_v, hslot_v, bidx_v, bslot_v, pidx_v, pslot_v,
                blk_v, ring_v, bcnt_s, bsem, rsem):
    wid = lax.axis_index("s") * NC + lax.axis_index("c")
    lo = wid * SPAN
    g0 = wid * BPT
    gmax = jnp.minimum(g0 + BPT, NBLK_TOT)
    iota = lax.iota(jnp.int32, L)

    for b0_ in range(NBUF - 1):
        @pl.when(g0 + b0_ < gmax)
        def _(b0_=b0_):
            off0 = pl.multiple_of((g0 + b0_) * BLK, BLK)
            pltpu.async_copy(tabT_hbm.at[:, pl.ds(off0, BLK)], blk_v.at[b0_], bsem)

    pltpu.sync_copy(lk_hbm, lk_v)

    def scan_body(i, cur):
        v = lk_v[pl.ds(i * L, L)]
        m = (v >= lo) & (v < lo + SPAN)
        n = plsc.all_reduce_population_count(m)[0]
        plsc.store_compressed(hidx_v.at[pl.ds(cur, L)], v - lo, mask=m)
        plsc.store_compressed(hslot_v.at[pl.ds(cur, L)], iota + i * L, mask=m)
        return jnp.minimum(cur + n, HCAP)

    nhits = lax.fori_loop(0, NLK // L, scan_body, jnp.int32(0))

    for k in range(NBKT):
        def bkt_body(h, cur, k=k):
            v = hidx_v[pl.ds(h * L, L)]
            s = hslot_v[pl.ds(h * L, L)]
            valid = (iota + h * L) < nhits
            m = valid & (lax.shift_right_logical(v, 11) == k)
            n = plsc.all_reduce_population_count(m)[0]
            plsc.store_compressed(bidx_v.at[k, pl.ds(cur, L)], v, mask=m)
            plsc.store_compressed(bslot_v.at[k, pl.ds(cur, L)], s, mask=m)
            return jnp.minimum(cur + n, BCAP)

        nb = lax.fori_loop(0, (HCAP + L - 1) // L, bkt_body, jnp.int32(0))
        bcnt_s[k] = nb

    def issue(g, b):
        @pl.when(g < gmax)
        def _():
            off = pl.multiple_of(g * BLK, BLK)
            pltpu.async_copy(tabT_hbm.at[:, pl.ds(off, BLK)], blk_v.at[b], bsem)

    def process(g, b, tot):
        lg = g - g0
        live = g < gmax

        @pl.when(live)
        def _():
            pltpu.make_async_copy(
                tabT_hbm.at[:, pl.ds(0, BLK)], blk_v.at[b], bsem).wait()

        k = lax.shift_right_logical(lg, 4)
        ck = bcnt_s[k]
        vlo = lg * BLK

        def ext_body(h, cur):
            v = bidx_v[k, pl.ds(h * L, L)]
            s = bslot_v[k, pl.ds(h * L, L)]
            valid = (iota + h * L) < ck
            m = valid & (v >= vlo) & (v < vlo + BLK) & jnp.full((L,), live)
            n = plsc.all_reduce_population_count(m)[0]
            plsc.store_compressed(pidx_v.at[pl.ds(cur, L)], v, mask=m)
            plsc.store_compressed(pslot_v.at[pl.ds(cur, L)], s, mask=m)
            return jnp.minimum(cur + n, PCAP)

        np_ = lax.fori_loop(0, (BCAP + L) // L, ext_body, jnp.int32(0))

        def hit_body(h, tot):
            idx = pidx_v[pl.ds(h, L)][0]
            slot = pslot_v[pl.ds(h, L)][0]
            c = jnp.broadcast_to(idx & (BLK - 1), (L,))
            rs = lax.rem(tot, NRING)

            @pl.when(tot >= NRING)
            def _():
                pltpu.make_async_copy(stage_hbm.at[pl.ds(0, D)], ring_v.at[rs], rsem).wait()

            for fg in range(D // L):
                ring_v[rs, pl.ds(fg * L, L)] = plsc.load_gather(
                    blk_v.at[b], [iota + fg * L, c])
            pltpu.async_copy(ring_v.at[rs], stage_hbm.at[pl.ds(slot * D, D)], rsem)
            return tot + 1

        return lax.fori_loop(0, np_, hit_body, tot)

    def quad_body(q, tot):
        for b in range(NBUF):
            g = g0 + q * NBUF + b
            tot = process(g, b, tot)
            issue(g + NBUF - 1, (b + NBUF - 1) % NBUF)
        return tot

    tot = lax.fori_loop(0, (BPT + NBUF - 1) // NBUF, quad_body, jnp.int32(0))

    def drain_body(j, _):
        @pl.when(j < jnp.minimum(tot, NRING))
        def _():
            pltpu.make_async_copy(stage_hbm.at[pl.ds(0, D)], ring_v.at[j], rsem).wait()
        return 0

    lax.fori_loop(0, NRING, drain_body, 0)


@functools.partial(
    pl.kernel,
    out_type=jax.ShapeDtypeStruct((B,), jnp.float32),
    mesh=_mesh,
    compiler_params=pltpu.CompilerParams(needs_layout_passes=False),
    scratch_types=[
        pltpu.VMEM((NCH, CHUNK), jnp.int32),
        pltpu.VMEM((NCH, CHUNK), jnp.int32),
        pltpu.VMEM((CHUNK * D,), jnp.float32),
        pltpu.VMEM((CHUNK * D,), jnp.float32),
        pltpu.VMEM((CHUNK,), jnp.float32),
        pltpu.VMEM((CHUNK,), jnp.float32),
        pltpu.VMEM((96,), jnp.float32),
        pltpu.VMEM((L * L,), jnp.float32),
        pltpu.VMEM((BPW,), jnp.float32),
        pltpu.SemaphoreType.DMA,
    ],
)
def _epilogue(stage_hbm, bias_hbm, ia_hbm, ib_hbm, w_hbm, out_hbm,
              ia_v, ib_v, ra_v, rb_v, ba_v, bb_v, w_v, m_v, o_v, sem):
    wid = lax.axis_index("s") * NC + lax.axis_index("c")
    base = wid * BPW
    pltpu.sync_copy(ia_hbm.at[wid], ia_v)
    pltpu.sync_copy(ib_hbm.at[wid], ib_v)
    pltpu.sync_copy(w_hbm, w_v)
    wv = [w_v[pl.ds(c * L, L)] for c in range(D // L)]
    w64v = w_v[pl.ds(D, L)]
    b0v = w_v[pl.ds(D + L, L)]
    iota = lax.iota(jnp.int32, L)

    for j in range(NCH):
        r0 = base + j * CHUNK
        cps = [
            pltpu.async_copy(stage_hbm.at[pl.ds(r0 * D, CHUNK * D)], ra_v, sem),
            pltpu.async_copy(stage_hbm.at[pl.ds((B + r0) * D, CHUNK * D)], rb_v, sem),
            pltpu.async_copy(bias_hbm.at[ia_v.at[j]], ba_v, sem),
            pltpu.async_copy(bias_hbm.at[ib_v.at[j]], bb_v, sem),
        ]
        for cp in cps:
            cp.wait()

        def group(g, _):
            rbase = g * L
            for r in range(L):
                row = rbase + r
                acc = (ra_v[pl.ds(row * D, L)]
                       * rb_v[pl.ds(row * D, L)] * wv[0])
                for c in range(1, D // L):
                    acc = acc + (ra_v[pl.ds(row * D + c * L, L)]
                                 * rb_v[pl.ds(row * D + c * L, L)] * wv[c])
                m_v[pl.ds(r * L, L)] = acc
            res = plsc.load_gather(m_v, [iota * L])
            for kcol in range(1, L):
                res = res + plsc.load_gather(m_v, [iota * L + kcol])
            x = res + (ba_v[pl.ds(rbase, L)] + bb_v[pl.ds(rbase, L)]) * w64v + b0v
            o_v[pl.ds(j * CHUNK + rbase, L)] = 1.0 / (1.0 + jnp.exp(-x))
            return 0

        lax.fori_loop(0, GROUPS, group, 0)

    pltpu.sync_copy(o_v, out_hbm.at[pl.ds(base, BPW)])


def kernel(drug_a, drug_b, emb_table, bias_table, dense_W, dense_b):
    ia32 = drug_a.astype(jnp.int32)
    ib32 = drug_b.astype(jnp.int32)
    lk = jnp.concatenate([ia32, ib32])
    stage = _scan_route(emb_table.T, lk)
    w = dense_W[:, 0]
    wpack = jnp.concatenate([
        w[:D],
        jnp.full((L,), w[D], jnp.float32),
        jnp.full((L,), dense_b[0], jnp.float32),
    ])
    out = _epilogue(stage, bias_table[:, 0],
                    ia32.reshape(NW, NCH, CHUNK), ib32.reshape(NW, NCH, CHUNK),
                    wpack)
    return out.reshape(B, 1)


# speedup vs baseline: 4.2857x; 1.0992x over previous
import functools

import jax
import jax.numpy as jnp
from jax import lax
from jax.experimental import pallas as pl
from jax.experimental.pallas import tpu as pltpu
from jax.experimental.pallas import tpu_sc as plsc

NC, NS, L = 2, 16, 16
NW = NC * NS
B = 16384
NLK = 2 * B
D = 64
NROW = 1000001
BLK = 128
NBLK_TOT = (NROW + BLK - 1) // BLK
BPT = (NBLK_TOT + NW - 1) // NW
SPAN = BPT * BLK
NBKT = 16
BKT_SPAN = 2048
HCAP = 2048
BCAP = 288
PCAP = 96
NRING = 8
NBUF = 4

BPW = B // NW
CHUNK = 128
NCH = BPW // CHUNK
GROUPS = CHUNK // L

_mesh = plsc.VectorSubcoreMesh(
    core_axis_name="c", subcore_axis_name="s", num_cores=NC, num_subcores=NS)


@functools.partial(
    pl.kernel,
    out_type=jax.ShapeDtypeStruct((NLK * D,), jnp.float32),
    mesh=_mesh,
    compiler_params=pltpu.CompilerParams(
        needs_layout_passes=False, use_tc_tiling_on_sc=True),
    scratch_types=[
        pltpu.VMEM((NLK,), jnp.int32),
        pltpu.VMEM((HCAP + L,), jnp.int32),
        pltpu.VMEM((HCAP + L,), jnp.int32),
        pltpu.VMEM((NBKT, BCAP + L), jnp.int32),
        pltpu.VMEM((NBKT, BCAP + L), jnp.int32),
        pltpu.VMEM((PCAP + L,), jnp.int32),
        pltpu.VMEM((PCAP + L,), jnp.int32),
        pltpu.VMEM((NBUF, D, BLK), jnp.float32),
        pltpu.VMEM((NRING, D), jnp.float32),
        pltpu.SMEM((NBKT,), jnp.int32),
        pltpu.SemaphoreType.DMA,
        pltpu.SemaphoreType.DMA,
    ],
)
def _scan_route(tabT_hbm, ia_hbm, ib_hbm, stage_hbm,
                lk_v, hidx_v, hslot_v, bidx_v, bslot_v, pidx_v, pslot_v,
                blk_v, ring_v, bcnt_s, bsem, rsem):
    wid = lax.axis_index("s") * NC + lax.axis_index("c")
    lo = wid * SPAN
    g0 = wid * BPT
    gmax = jnp.minimum(g0 + BPT, NBLK_TOT)
    iota = lax.iota(jnp.int32, L)

    for b0_ in range(NBUF - 1):
        @pl.when(g0 + b0_ < gmax)
        def _(b0_=b0_):
            off0 = pl.multiple_of((g0 + b0_) * BLK, BLK)
            pltpu.async_copy(tabT_hbm.at[:, pl.ds(off0, BLK)], blk_v.at[b0_], bsem)

    pltpu.sync_copy(ia_hbm, lk_v.at[pl.ds(0, B)])
    pltpu.sync_copy(ib_hbm, lk_v.at[pl.ds(B, B)])

    def scan_body(i, cur):
        v = lk_v[pl.ds(i * L, L)]
        m = (v >= lo) & (v < lo + SPAN)
        n = plsc.all_reduce_population_count(m)[0]
        plsc.store_compressed(hidx_v.at[pl.ds(cur, L)], v - lo, mask=m)
        plsc.store_compressed(hslot_v.at[pl.ds(cur, L)], iota + i * L, mask=m)
        return jnp.minimum(cur + n, HCAP)

    nhits = lax.fori_loop(0, NLK // L, scan_body, jnp.int32(0))

    for k in range(NBKT):
        def bkt_body(h, cur, k=k):
            v = hidx_v[pl.ds(h * L, L)]
            s = hslot_v[pl.ds(h * L, L)]
            valid = (iota + h * L) < nhits
            m = valid & (lax.shift_right_logical(v, 11) == k)
            n = plsc.all_reduce_population_count(m)[0]
            plsc.store_compressed(bidx_v.at[k, pl.ds(cur, L)], v, mask=m)
            plsc.store_compressed(bslot_v.at[k, pl.ds(cur, L)], s, mask=m)
            return jnp.minimum(cur + n, BCAP)

        nb = lax.fori_loop(0, lax.shift_right_logical(nhits + L - 1, 4),
                           bkt_body, jnp.int32(0))
        bcnt_s[k] = nb

    def issue(g, b):
        @pl.when(g < gmax)
        def _():
            off = pl.multiple_of(g * BLK, BLK)
            pltpu.async_copy(tabT_hbm.at[:, pl.ds(off, BLK)], blk_v.at[b], bsem)

    def process(g, b, tot):
        lg = g - g0
        live = g < gmax

        @pl.when(live)
        def _():
            pltpu.make_async_copy(
                tabT_hbm.at[:, pl.ds(0, BLK)], blk_v.at[b], bsem).wait()

        k = lax.shift_right_logical(lg, 4)
        ck = bcnt_s[k]
        vlo = lg * BLK

        def ext_body(h, cur):
            v = bidx_v[k, pl.ds(h * L, L)]
            s = bslot_v[k, pl.ds(h * L, L)]
            valid = (iota + h * L) < ck
            m = valid & (v >= vlo) & (v < vlo + BLK) & jnp.full((L,), live)
            n = plsc.all_reduce_population_count(m)[0]
            plsc.store_compressed(pidx_v.at[pl.ds(cur, L)], v, mask=m)
            plsc.store_compressed(pslot_v.at[pl.ds(cur, L)], s, mask=m)
            return jnp.minimum(cur + n, PCAP)

        np_ = lax.fori_loop(0, lax.shift_right_logical(ck + L - 1, 4),
                            ext_body, jnp.int32(0))

        def hit_body(h, tot):
            idx = pidx_v[pl.ds(h, L)][0]
            slot = pslot_v[pl.ds(h, L)][0]
            c = jnp.broadcast_to(idx & (BLK - 1), (L,))
            rs = lax.rem(tot, NRING)

            @pl.when(tot >= NRING)
            def _():
                pltpu.make_async_copy(stage_hbm.at[pl.ds(0, D)], ring_v.at[rs], rsem).wait()

            for fg in range(D // L):
                ring_v[rs, pl.ds(fg * L, L)] = plsc.load_gather(
                    blk_v.at[b], [iota + fg * L, c])
            pltpu.async_copy(ring_v.at[rs], stage_hbm.at[pl.ds(slot * D, D)], rsem)
            return tot + 1

        return lax.fori_loop(0, np_, hit_body, tot)

    def quad_body(q, tot):
        for b in range(NBUF):
            g = g0 + q * NBUF + b
            tot = process(g, b, tot)
            issue(g + NBUF - 1, (b + NBUF - 1) % NBUF)
        return tot

    tot = lax.fori_loop(0, (BPT + NBUF - 1) // NBUF, quad_body, jnp.int32(0))

    def drain_body(j, _):
        @pl.when(j < jnp.minimum(tot, NRING))
        def _():
            pltpu.make_async_copy(stage_hbm.at[pl.ds(0, D)], ring_v.at[j], rsem).wait()
        return 0

    lax.fori_loop(0, NRING, drain_body, 0)


@functools.partial(
    pl.kernel,
    out_type=jax.ShapeDtypeStruct((B,), jnp.float32),
    mesh=_mesh,
    compiler_params=pltpu.CompilerParams(needs_layout_passes=False),
    scratch_types=[
        pltpu.VMEM((NCH, CHUNK), jnp.int32),
        pltpu.VMEM((NCH, CHUNK), jnp.int32),
        pltpu.VMEM((CHUNK * D,), jnp.float32),
        pltpu.VMEM((CHUNK * D,), jnp.float32),
        pltpu.VMEM((CHUNK,), jnp.float32),
        pltpu.VMEM((CHUNK,), jnp.float32),
        pltpu.VMEM((96,), jnp.float32),
        pltpu.VMEM((L * L,), jnp.float32),
        pltpu.VMEM((BPW,), jnp.float32),
        pltpu.SemaphoreType.DMA,
    ],
)
def _epilogue(stage_hbm, bias_hbm, ia_hbm, ib_hbm, w_hbm, out_hbm,
              ia_v, ib_v, ra_v, rb_v, ba_v, bb_v, w_v, m_v, o_v, sem):
    wid = lax.axis_index("s") * NC + lax.axis_index("c")
    base = wid * BPW
    pltpu.sync_copy(ia_hbm.at[wid], ia_v)
    pltpu.sync_copy(ib_hbm.at[wid], ib_v)
    pltpu.sync_copy(w_hbm, w_v)
    wv = [w_v[pl.ds(c * L, L)] for c in range(D // L)]
    w64v = w_v[pl.ds(D, L)]
    b0v = w_v[pl.ds(D + L, L)]
    iota = lax.iota(jnp.int32, L)

    for j in range(NCH):
        r0 = base + j * CHUNK
        cps = [
            pltpu.async_copy(stage_hbm.at[pl.ds(r0 * D, CHUNK * D)], ra_v, sem),
            pltpu.async_copy(stage_hbm.at[pl.ds((B + r0) * D, CHUNK * D)], rb_v, sem),
            pltpu.async_copy(bias_hbm.at[ia_v.at[j]], ba_v, sem),
            pltpu.async_copy(bias_hbm.at[ib_v.at[j]], bb_v, sem),
        ]
        for cp in cps:
            cp.wait()

        def group(g, _):
            rbase = g * L
            for r in range(L):
                row = rbase + r
                acc = (ra_v[pl.ds(row * D, L)]
                       * rb_v[pl.ds(row * D, L)] * wv[0])
                for c in range(1, D // L):
                    acc = acc + (ra_v[pl.ds(row * D + c * L, L)]
                                 * rb_v[pl.ds(row * D + c * L, L)] * wv[c])
                m_v[pl.ds(r * L, L)] = acc
            res = plsc.load_gather(m_v, [iota * L])
            for kcol in range(1, L):
                res = res + plsc.load_gather(m_v, [iota * L + kcol])
            x = res + (ba_v[pl.ds(rbase, L)] + bb_v[pl.ds(rbase, L)]) * w64v + b0v
            o_v[pl.ds(j * CHUNK + rbase, L)] = 1.0 / (1.0 + jnp.exp(-x))
            return 0

        lax.fori_loop(0, GROUPS, group, 0)

    pltpu.sync_copy(o_v, out_hbm.at[pl.ds(base, BPW)])


def kernel(drug_a, drug_b, emb_table, bias_table, dense_W, dense_b):
    ia32 = drug_a.astype(jnp.int32)
    ib32 = drug_b.astype(jnp.int32)
    stage = _scan_route(emb_table.T, ia32, ib32)
    w = dense_W[:, 0]
    wpack = jnp.concatenate([
        w[:D],
        jnp.full((L,), w[D], jnp.float32),
        jnp.full((L,), dense_b[0], jnp.float32),
    ])
    out = _epilogue(stage, bias_table[:, 0],
                    ia32.reshape(NW, NCH, CHUNK), ib32.reshape(NW, NCH, CHUNK),
                    wpack)
    return out.reshape(B, 1)


# speedup vs baseline: 4.9480x; 1.1545x over previous
import functools

import jax
import jax.numpy as jnp
from jax import lax
from jax.experimental import pallas as pl
from jax.experimental.pallas import tpu as pltpu
from jax.experimental.pallas import tpu_sc as plsc

NC, NS, L = 2, 16, 16
NW = NC * NS
B = 16384
NLK = 2 * B
D = 64
NROW = 1000001
BLK = 256
NBLK_TOT = (NROW + BLK - 1) // BLK
BPT = (NBLK_TOT + NW - 1) // NW
SPAN = BPT * BLK
NBKT = 16
BKT_SPAN = 2048
HCAP = 2048
BCAP = 288
PCAP = 96
NRING = 8
NBUF = 4

BPW = B // NW
CHUNK = 128
NCH = BPW // CHUNK
GROUPS = CHUNK // L

_mesh = plsc.VectorSubcoreMesh(
    core_axis_name="c", subcore_axis_name="s", num_cores=NC, num_subcores=NS)


@functools.partial(
    pl.kernel,
    out_type=jax.ShapeDtypeStruct((NLK * D,), jnp.float32),
    mesh=_mesh,
    compiler_params=pltpu.CompilerParams(
        needs_layout_passes=False, use_tc_tiling_on_sc=True),
    scratch_types=[
        pltpu.VMEM((NLK,), jnp.int32),
        pltpu.VMEM((HCAP + L,), jnp.int32),
        pltpu.VMEM((HCAP + L,), jnp.int32),
        pltpu.VMEM((NBKT, BCAP + L), jnp.int32),
        pltpu.VMEM((NBKT, BCAP + L), jnp.int32),
        pltpu.VMEM((PCAP + L,), jnp.int32),
        pltpu.VMEM((PCAP + L,), jnp.int32),
        pltpu.VMEM((NBUF, D, BLK), jnp.float32),
        pltpu.VMEM((NRING, D), jnp.float32),
        pltpu.SMEM((NBKT,), jnp.int32),
        pltpu.SemaphoreType.DMA,
        pltpu.SemaphoreType.DMA,
    ],
)
def _scan_route(tabT_hbm, ia_hbm, ib_hbm, stage_hbm,
                lk_v, hidx_v, hslot_v, bidx_v, bslot_v, pidx_v, pslot_v,
                blk_v, ring_v, bcnt_s, bsem, rsem):
    wid = lax.axis_index("s") * NC + lax.axis_index("c")
    lo = wid * SPAN
    g0 = wid * BPT
    gmax = jnp.minimum(g0 + BPT, NBLK_TOT)
    iota = lax.iota(jnp.int32, L)

    for b0_ in range(NBUF - 1):
        @pl.when(g0 + b0_ < gmax)
        def _(b0_=b0_):
            off0 = pl.multiple_of((g0 + b0_) * BLK, BLK)
            pltpu.async_copy(tabT_hbm.at[:, pl.ds(off0, BLK)], blk_v.at[b0_], bsem)

    pltpu.sync_copy(ia_hbm, lk_v.at[pl.ds(0, B)])
    pltpu.sync_copy(ib_hbm, lk_v.at[pl.ds(B, B)])

    def scan_body(i, cur):
        v = lk_v[pl.ds(i * L, L)]
        m = (v >= lo) & (v < lo + SPAN)
        n = plsc.all_reduce_population_count(m)[0]
        plsc.store_compressed(hidx_v.at[pl.ds(cur, L)], v - lo, mask=m)
        plsc.store_compressed(hslot_v.at[pl.ds(cur, L)], iota + i * L, mask=m)
        return jnp.minimum(cur + n, HCAP)

    nhits = lax.fori_loop(0, NLK // L, scan_body, jnp.int32(0))

    for k in range(NBKT):
        def bkt_body(h, cur, k=k):
            v = hidx_v[pl.ds(h * L, L)]
            s = hslot_v[pl.ds(h * L, L)]
            valid = (iota + h * L) < nhits
            m = valid & (lax.shift_right_logical(v, 11) == k)
            n = plsc.all_reduce_population_count(m)[0]
            plsc.store_compressed(bidx_v.at[k, pl.ds(cur, L)], v, mask=m)
            plsc.store_compressed(bslot_v.at[k, pl.ds(cur, L)], s, mask=m)
            return jnp.minimum(cur + n, BCAP)

        nb = lax.fori_loop(0, lax.shift_right_logical(nhits + L - 1, 4),
                           bkt_body, jnp.int32(0))
        bcnt_s[k] = nb

    def issue(g, b):
        @pl.when(g < gmax)
        def _():
            off = pl.multiple_of(g * BLK, BLK)
            pltpu.async_copy(tabT_hbm.at[:, pl.ds(off, BLK)], blk_v.at[b], bsem)

    def process(g, b, tot):
        lg = g - g0
        live = g < gmax

        @pl.when(live)
        def _():
            pltpu.make_async_copy(
                tabT_hbm.at[:, pl.ds(0, BLK)], blk_v.at[b], bsem).wait()

        k = lax.shift_right_logical(lg, 3)
        ck = bcnt_s[k]
        vlo = lg * BLK

        def ext_body(h, cur):
            v = bidx_v[k, pl.ds(h * L, L)]
            s = bslot_v[k, pl.ds(h * L, L)]
            valid = (iota + h * L) < ck
            m = valid & (v >= vlo) & (v < vlo + BLK) & jnp.full((L,), live)
            n = plsc.all_reduce_population_count(m)[0]
            plsc.store_compressed(pidx_v.at[pl.ds(cur, L)], v, mask=m)
            plsc.store_compressed(pslot_v.at[pl.ds(cur, L)], s, mask=m)
            return jnp.minimum(cur + n, PCAP)

        np_ = lax.fori_loop(0, lax.shift_right_logical(ck + L - 1, 4),
                            ext_body, jnp.int32(0))

        def hit_body(h, tot):
            idx = pidx_v[pl.ds(h, L)][0]
            slot = pslot_v[pl.ds(h, L)][0]
            c = jnp.broadcast_to(idx & (BLK - 1), (L,))
            rs = lax.rem(tot, NRING)

            @pl.when(tot >= NRING)
            def _():
                pltpu.make_async_copy(stage_hbm.at[pl.ds(0, D)], ring_v.at[rs], rsem).wait()

            for fg in range(D // L):
                ring_v[rs, pl.ds(fg * L, L)] = plsc.load_gather(
                    blk_v.at[b], [iota + fg * L, c])
            pltpu.async_copy(ring_v.at[rs], stage_hbm.at[pl.ds(slot * D, D)], rsem)
            return tot + 1

        return lax.fori_loop(0, np_, hit_body, tot)

    def quad_body(q, tot):
        for b in range(NBUF):
            g = g0 + q * NBUF + b
            tot = process(g, b, tot)
            issue(g + NBUF - 1, (b + NBUF - 1) % NBUF)
        return tot

    tot = lax.fori_loop(0, (BPT + NBUF - 1) // NBUF, quad_body, jnp.int32(0))

    def drain_body(j, _):
        @pl.when(j < jnp.minimum(tot, NRING))
        def _():
            pltpu.make_async_copy(stage_hbm.at[pl.ds(0, D)], ring_v.at[j], rsem).wait()
        return 0

    lax.fori_loop(0, NRING, drain_body, 0)


@functools.partial(
    pl.kernel,
    out_type=jax.ShapeDtypeStruct((B,), jnp.float32),
    mesh=_mesh,
    compiler_params=pltpu.CompilerParams(needs_layout_passes=False),
    scratch_types=[
        pltpu.VMEM((BPW,), jnp.int32),
        pltpu.VMEM((BPW,), jnp.int32),
        pltpu.VMEM((CHUNK * D,), jnp.float32),
        pltpu.VMEM((CHUNK * D,), jnp.float32),
        pltpu.VMEM((CHUNK,), jnp.float32),
        pltpu.VMEM((CHUNK,), jnp.float32),
        pltpu.VMEM((96,), jnp.float32),
        pltpu.VMEM((L * L,), jnp.float32),
        pltpu.VMEM((BPW,), jnp.float32),
        pltpu.SemaphoreType.DMA,
    ],
)
def _epilogue(stage_hbm, bias_hbm, ia_hbm, ib_hbm, w_hbm, out_hbm,
              ia_v, ib_v, ra_v, rb_v, ba_v, bb_v, w_v, m_v, o_v, sem):
    wid = lax.axis_index("s") * NC + lax.axis_index("c")
    base = wid * BPW
    pltpu.sync_copy(ia_hbm.at[pl.ds(base, BPW)], ia_v)
    pltpu.sync_copy(ib_hbm.at[pl.ds(base, BPW)], ib_v)
    pltpu.sync_copy(w_hbm, w_v)
    wv = [w_v[pl.ds(c * L, L)] for c in range(D // L)]
    w64v = w_v[pl.ds(D, L)]
    b0v = w_v[pl.ds(D + L, L)]
    iota = lax.iota(jnp.int32, L)

    for j in range(NCH):
        r0 = base + j * CHUNK
        cps = [
            pltpu.async_copy(stage_hbm.at[pl.ds(r0 * D, CHUNK * D)], ra_v, sem),
            pltpu.async_copy(stage_hbm.at[pl.ds((B + r0) * D, CHUNK * D)], rb_v, sem),
            pltpu.async_copy(bias_hbm.at[ia_v.at[pl.ds(j * CHUNK, CHUNK)]], ba_v, sem),
            pltpu.async_copy(bias_hbm.at[ib_v.at[pl.ds(j * CHUNK, CHUNK)]], bb_v, sem),
        ]
        for cp in cps:
            cp.wait()

        def group(g, _):
            rbase = g * L
            for r in range(L):
                row = rbase + r
                acc = (ra_v[pl.ds(row * D, L)]
                       * rb_v[pl.ds(row * D, L)] * wv[0])
                for c in range(1, D // L):
                    acc = acc + (ra_v[pl.ds(row * D + c * L, L)]
                                 * rb_v[pl.ds(row * D + c * L, L)] * wv[c])
                m_v[pl.ds(r * L, L)] = acc
            res = plsc.load_gather(m_v, [iota * L])
            for kcol in range(1, L):
                res = res + plsc.load_gather(m_v, [iota * L + kcol])
            x = res + (ba_v[pl.ds(rbase, L)] + bb_v[pl.ds(rbase, L)]) * w64v + b0v
            o_v[pl.ds(j * CHUNK + rbase, L)] = 1.0 / (1.0 + jnp.exp(-x))
            return 0

        lax.fori_loop(0, GROUPS, group, 0)

    pltpu.sync_copy(o_v, out_hbm.at[pl.ds(base, BPW)])


def kernel(drug_a, drug_b, emb_table, bias_table, dense_W, dense_b):
    ia32 = drug_a.astype(jnp.int32)
    ib32 = drug_b.astype(jnp.int32)
    stage = _scan_route(emb_table.T, ia32, ib32)
    w = dense_W[:, 0]
    wpack = jnp.concatenate([
        w[:D],
        jnp.full((L,), w[D], jnp.float32),
        jnp.full((L,), dense_b[0], jnp.float32),
    ])
    out = _epilogue(stage, bias_table[:, 0], ia32, ib32, wpack)
    return out.reshape(B, 1)
